# baseline (speedup 1.0000x reference)
import jax
import jax.numpy as jnp
from jax.experimental import pallas as pl


_ROW_BLOCK = 10000


def _mlp_kernel(h_ref, w1_ref, b1_ref, w2_ref, b2_ref, o_ref):
    hidden = jnp.dot(h_ref[...].astype(jnp.bfloat16),
                     w1_ref[...].astype(jnp.bfloat16),
                     preferred_element_type=jnp.float32)
    hidden = jnp.maximum(hidden + b1_ref[...], 0.0)
    out = jnp.dot(hidden.astype(jnp.bfloat16),
                  w2_ref[...].astype(jnp.bfloat16),
                  preferred_element_type=jnp.float32)
    o_ref[...] = out + b2_ref[...]


def kernel(h, edge_index, W1, b1, W2, b2):
    del edge_index
    n, d_in = h.shape
    d_hid = W1.shape[1]
    d_out = W2.shape[1]
    w1 = W1[:d_in]
    w2 = W2[:d_hid]
    grid = (pl.cdiv(n, _ROW_BLOCK),)
    return pl.pallas_call(
        _mlp_kernel,
        grid=grid,
        in_specs=[
            pl.BlockSpec((_ROW_BLOCK, d_in), lambda i: (i, 0)),
            pl.BlockSpec((d_in, d_hid), lambda i: (0, 0)),
            pl.BlockSpec((1, d_hid), lambda i: (0, 0)),
            pl.BlockSpec((d_hid, d_out), lambda i: (0, 0)),
            pl.BlockSpec((1, d_out), lambda i: (0, 0)),
        ],
        out_specs=pl.BlockSpec((_ROW_BLOCK, d_out), lambda i: (i, 0)),
        out_shape=jax.ShapeDtypeStruct((n, d_out), h.dtype),
    )(h, w1, b1.reshape(1, d_hid), w2, b2.reshape(1, d_out))


# speedup vs baseline: 1.1441x; 1.1441x over previous
import jax
import jax.numpy as jnp
from jax.experimental import pallas as pl


_ROW_BLOCK = 10000


def _mlp_kernel(h_ref, w1_ref, b1_ref, w2_ref, b2_ref, o_ref):
    hidden = jnp.dot(h_ref[...], w1_ref[...], preferred_element_type=jnp.float32)
    hidden = jnp.maximum(hidden + b1_ref[...], 0.0)
    out = jnp.dot(hidden, w2_ref[...], preferred_element_type=jnp.float32)
    o_ref[...] = out + b2_ref[...]


def kernel(h, edge_index, W1, b1, W2, b2):
    del edge_index
    n, d_in = h.shape
    d_hid = W1.shape[1]
    d_out = W2.shape[1]
    w1 = W1[:d_in]
    w2 = W2[:d_hid]
    grid = (pl.cdiv(n, _ROW_BLOCK),)
    return pl.pallas_call(
        _mlp_kernel,
        grid=grid,
        in_specs=[
            pl.BlockSpec((_ROW_BLOCK, d_in), lambda i: (i, 0)),
            pl.BlockSpec((d_in, d_hid), lambda i: (0, 0)),
            pl.BlockSpec((1, d_hid), lambda i: (0, 0)),
            pl.BlockSpec((d_hid, d_out), lambda i: (0, 0)),
            pl.BlockSpec((1, d_out), lambda i: (0, 0)),
        ],
        out_specs=pl.BlockSpec((_ROW_BLOCK, d_out), lambda i: (i, 0)),
        out_shape=jax.ShapeDtypeStruct((n, d_out), h.dtype),
    )(h, w1, b1.reshape(1, d_hid), w2, b2.reshape(1, d_out))
